# baseline (speedup 1.0000x reference)
import functools

import jax
import jax.numpy as jnp
from jax import lax
from jax.experimental import pallas as pl
from jax.experimental.pallas import tpu as pltpu
from jax.experimental.pallas import tpu_sc as plsc

NC, NS = 2, 16
NW = NC * NS
D = 64
LANES = 16


def _iota16():
    return lax.iota(jnp.int32, LANES)


def _transpose_win(in_ref, out_ref, nq):
    iot = _iota16()
    for r0 in range(0, D, 16):
        rowv = r0 + iot
        for k in range(16):
            perm = jnp.bitwise_and(iot + k, 15)
            ocol = lax.shift_left(jnp.bitwise_and(perm, 1), 6) + rowv

            @plsc.parallel_loop(0, 2 * nq, step=16)
            def _c(c0, rowv=rowv, perm=perm, ocol=ocol):
                c_vec = c0 + perm
                vals = plsc.load_gather(in_ref, [rowv, c_vec])
                q_vec = lax.shift_right_logical(c_vec, 1)
                plsc.store_scatter(out_ref, [q_vec, ocol], vals)


@functools.lru_cache(maxsize=None)
def _make_repack(V):
    PAIRS = V // 2
    WFULL = V // 128
    WREM = V - WFULL * 128
    KMAIN = WFULL // NW
    NEXTRA = WFULL - KMAIN * NW
    assert KMAIN % 2 == 0 and KMAIN >= 4 and WREM in (0, 64)
    mesh = plsc.VectorSubcoreMesh(core_axis_name="c", subcore_axis_name="s")

    @functools.partial(
        pl.kernel,
        out_type=jax.ShapeDtypeStruct((PAIRS, 128), jnp.float32),
        mesh=mesh,
        compiler_params=pltpu.CompilerParams(needs_layout_passes=False),
        scratch_types=[
            pltpu.VMEM((2, D, 128), jnp.float32),
            pltpu.VMEM((2, D, 128), jnp.float32),
            pltpu.SemaphoreType.DMA((2,)),
            pltpu.SemaphoreType.DMA((2,)),
        ],
    )
    def repack_kernel(tabT_hbm, tail_hbm, tabp_hbm, in_v, out_v, rsem, wsem):
        wid = lax.axis_index("s") * NC + lax.axis_index("c")

        def win_of(k):
            return wid + NW * k

        def fire_read(k, j):
            pltpu.async_copy(tabT_hbm.at[:, pl.ds(win_of(k) * 128, 128)],
                             in_v.at[j], rsem.at[j])

        def wait_read(j):
            pltpu.make_async_copy(tabT_hbm.at[:, pl.ds(0, 128)], in_v.at[j],
                                  rsem.at[j]).wait()

        def fire_write(k, j):
            pltpu.async_copy(out_v.at[j],
                             tabp_hbm.at[pl.ds(win_of(k) * 64, 64)],
                             wsem.at[j])

        def wait_write(j):
            pltpu.make_async_copy(out_v.at[j], tabp_hbm.at[pl.ds(0, 64)],
                                  wsem.at[j]).wait()

        nk = KMAIN + jnp.where(wid < NEXTRA, 1, 0)

        fire_read(0, 0)

        @pl.loop(0, nk)
        def _k(k):
            j = k % 2
            wait_read(j)

            @pl.when(k + 1 < nk)
            def _pf():
                fire_read(k + 1, 1 - j)

            @pl.when(k >= 2)
            def _ww():
                wait_write(j)

            _transpose_win(in_v.at[j], out_v.at[j], 64)
            fire_write(k, j)

        wait_write((nk - 2) % 2)
        wait_write((nk - 1) % 2)

        if WREM:
            @pl.when(wid == NEXTRA)
            def _rem():
                pltpu.sync_copy(tail_hbm, out_v.at[0, pl.ds(0, WREM // 2)])
                pltpu.sync_copy(out_v.at[0, pl.ds(0, WREM // 2)],
                                tabp_hbm.at[pl.ds(WFULL * 64, WREM // 2)])

    return repack_kernel


@functools.lru_cache(maxsize=None)
def _make_gather(T, BT, PAIRS):
    BW = BT // NW
    assert BW == 128 and T % 2 == 0 and T >= 6
    mesh = plsc.VectorSubcoreMesh(core_axis_name="c", subcore_axis_name="s")

    @functools.partial(
        pl.kernel,
        out_type=jax.ShapeDtypeStruct((T, D, BT), jnp.float32),
        mesh=mesh,
        compiler_params=pltpu.CompilerParams(needs_layout_passes=False),
        scratch_types=[
            pltpu.VMEM((T, BW), jnp.int32),
            pltpu.VMEM((T, BW), jnp.int32),
            pltpu.VMEM((2, BW, 128), jnp.float32),
            pltpu.VMEM((2, D, BW), jnp.float32),
            pltpu.SemaphoreType.DMA((2,)),
            pltpu.SemaphoreType.DMA((2,)),
        ],
    )
    def gather_kernel(tabp_hbm, idxT_hbm, out_hbm, idx_v, p_v, pairs_v,
                      face_v, gsem, wsem):
        wid = lax.axis_index("s") * NC + lax.axis_index("c")
        b0 = wid * BW
        pltpu.sync_copy(idxT_hbm.at[:, pl.ds(b0, BW)], idx_v)

        @pl.loop(0, T)
        def _pair_ids(t):
            for cb in range(8):
                sl = pl.ds(16 * cb, LANES)
                p_v[t, sl] = lax.shift_right_logical(idx_v[t, sl], 1)

        def fire_gather(t, j):
            pltpu.async_copy(tabp_hbm.at[p_v.at[t]], pairs_v.at[j],
                             gsem.at[j])

        def wait_gather(j):
            pltpu.make_async_copy(tabp_hbm.at[pl.ds(0, BW)], pairs_v.at[j],
                                  gsem.at[j]).wait()

        def fire_write(t, j):
            pltpu.async_copy(face_v.at[j], out_hbm.at[t, :, pl.ds(b0, BW)],
                             wsem.at[j])

        def wait_write(j):
            pltpu.make_async_copy(face_v.at[j], out_hbm.at[0, :, pl.ds(0, BW)],
                                  wsem.at[j]).wait()

        def compact(t, j):
            iot = _iota16()
            for tb in range(0, BW, 16):
                rowv = tb + iot
                h64 = lax.shift_left(
                    jnp.bitwise_and(idx_v[t, pl.ds(tb, LANES)], 1), 6)
                for k in range(16):
                    perm = jnp.bitwise_and(iot + k, 15)
                    colp = h64 + perm

                    @plsc.parallel_loop(0, D, step=16)
                    def _d(d0, rowv=rowv, perm=perm, colp=colp):
                        vals = plsc.load_gather(pairs_v.at[j],
                                                [rowv, colp + d0])
                        plsc.store_scatter(face_v.at[j],
                                           [perm + d0, rowv], vals)

        fire_gather(0, 0)

        @pl.loop(0, T)
        def _t(t):
            j = t % 2
            wait_gather(j)

            @pl.when(t + 1 < T)
            def _pf():
                fire_gather(t + 1, 1 - j)

            @pl.when(t >= 2)
            def _ww():
                wait_write(j)

            compact(t, j)
            fire_write(t, j)

        wait_write(0)
        wait_write(1)

    return gather_kernel


def kernel(indices, embed_weight):
    bt, t = indices.shape
    v, d = embed_weight.shape
    tab_t = jnp.transpose(embed_weight)
    idx_t = jnp.transpose(indices)
    wfull = v // 128
    tail = embed_weight[wfull * 128:].reshape(-1, 2 * d)
    tabp = _make_repack(v)(tab_t, tail)
    out_t = _make_gather(t, bt, v // 2)(tabp, idx_t)
    return jnp.transpose(out_t, (2, 0, 1))


# speedup vs baseline: 1.9737x; 1.9737x over previous
import functools

import jax
import jax.numpy as jnp
from jax import lax
from jax.experimental import pallas as pl
from jax.experimental.pallas import tpu as pltpu
from jax.experimental.pallas import tpu_sc as plsc

NC, NS = 2, 16
NW = NC * NS
D = 64
LANES = 16


def _iota16():
    return lax.iota(jnp.int32, LANES)


def _transpose_win(in_ref, out_ref, nq):
    iot = _iota16()
    for r0 in range(0, D, 16):
        rowv = r0 + iot

        @plsc.parallel_loop(0, 2 * nq, step=16)
        def _c(c0, rowv=rowv, r0=r0):
            for k in range(16):
                c_vec = c0 + jnp.bitwise_and(iot + k, 15)
                vals = plsc.load_gather(in_ref, [rowv, c_vec])
                q_vec = lax.shift_right_logical(c_vec, 1)
                ocol = lax.shift_left(jnp.bitwise_and(c_vec, 1), 6) + rowv
                plsc.store_scatter(out_ref, [q_vec, ocol], vals)


@functools.lru_cache(maxsize=None)
def _make_repack(V):
    PAIRS = V // 2
    WFULL = V // 128
    WREM = V - WFULL * 128
    KMAIN = WFULL // NW
    NEXTRA = WFULL - KMAIN * NW
    assert KMAIN % 2 == 0 and KMAIN >= 4 and WREM in (0, 64)
    mesh = plsc.VectorSubcoreMesh(core_axis_name="c", subcore_axis_name="s")

    @functools.partial(
        pl.kernel,
        out_type=jax.ShapeDtypeStruct((PAIRS, 128), jnp.float32),
        mesh=mesh,
        compiler_params=pltpu.CompilerParams(needs_layout_passes=False),
        scratch_types=[
            pltpu.VMEM((2, D, 128), jnp.float32),
            pltpu.VMEM((2, D, 128), jnp.float32),
            pltpu.SemaphoreType.DMA((2,)),
            pltpu.SemaphoreType.DMA((2,)),
        ],
    )
    def repack_kernel(tabT_hbm, tail_hbm, tabp_hbm, in_v, out_v, rsem, wsem):
        wid = lax.axis_index("s") * NC + lax.axis_index("c")

        def win_of(k):
            return wid + NW * k

        def fire_read(k, j):
            pltpu.async_copy(tabT_hbm.at[:, pl.ds(win_of(k) * 128, 128)],
                             in_v.at[j], rsem.at[j])

        def wait_read(j):
            pltpu.make_async_copy(tabT_hbm.at[:, pl.ds(0, 128)], in_v.at[j],
                                  rsem.at[j]).wait()

        def fire_write(k, j):
            pltpu.async_copy(out_v.at[j],
                             tabp_hbm.at[pl.ds(win_of(k) * 64, 64)],
                             wsem.at[j])

        def wait_write(j):
            pltpu.make_async_copy(out_v.at[j], tabp_hbm.at[pl.ds(0, 64)],
                                  wsem.at[j]).wait()

        nk = KMAIN + jnp.where(wid < NEXTRA, 1, 0)

        fire_read(0, 0)

        @pl.loop(0, nk)
        def _k(k):
            j = k % 2
            wait_read(j)

            @pl.when(k + 1 < nk)
            def _pf():
                fire_read(k + 1, 1 - j)

            @pl.when(k >= 2)
            def _ww():
                wait_write(j)

            _transpose_win(in_v.at[j], out_v.at[j], 64)
            fire_write(k, j)

        wait_write((nk - 2) % 2)
        wait_write((nk - 1) % 2)

        if WREM:
            @pl.when(wid == NEXTRA)
            def _rem():
                pltpu.sync_copy(tail_hbm, out_v.at[0, pl.ds(0, WREM // 2)])
                pltpu.sync_copy(out_v.at[0, pl.ds(0, WREM // 2)],
                                tabp_hbm.at[pl.ds(WFULL * 64, WREM // 2)])

    return repack_kernel


@functools.lru_cache(maxsize=None)
def _make_gather(T, BT, PAIRS):
    BW = BT // NW
    assert BW == 128 and T % 2 == 0 and T >= 6
    mesh = plsc.VectorSubcoreMesh(core_axis_name="c", subcore_axis_name="s")

    @functools.partial(
        pl.kernel,
        out_type=jax.ShapeDtypeStruct((T, D, BT), jnp.float32),
        mesh=mesh,
        compiler_params=pltpu.CompilerParams(needs_layout_passes=False),
        scratch_types=[
            pltpu.VMEM((T, BW), jnp.int32),
            pltpu.VMEM((T, BW), jnp.int32),
            pltpu.VMEM((2, BW, 128), jnp.float32),
            pltpu.VMEM((2, D, BW), jnp.float32),
            pltpu.SemaphoreType.DMA((2,)),
            pltpu.SemaphoreType.DMA((2,)),
        ],
    )
    def gather_kernel(tabp_hbm, idxT_hbm, out_hbm, idx_v, p_v, pairs_v,
                      face_v, gsem, wsem):
        wid = lax.axis_index("s") * NC + lax.axis_index("c")
        b0 = wid * BW
        pltpu.sync_copy(idxT_hbm.at[:, pl.ds(b0, BW)], idx_v)

        @pl.loop(0, T)
        def _pair_ids(t):
            for cb in range(8):
                sl = pl.ds(16 * cb, LANES)
                p_v[t, sl] = lax.shift_right_logical(idx_v[t, sl], 1)

        def fire_gather(t, j):
            pltpu.async_copy(tabp_hbm.at[p_v.at[t]], pairs_v.at[j],
                             gsem.at[j])

        def wait_gather(j):
            pltpu.make_async_copy(tabp_hbm.at[pl.ds(0, BW)], pairs_v.at[j],
                                  gsem.at[j]).wait()

        def fire_write(t, j):
            pltpu.async_copy(face_v.at[j], out_hbm.at[t, :, pl.ds(b0, BW)],
                             wsem.at[j])

        def wait_write(j):
            pltpu.make_async_copy(face_v.at[j], out_hbm.at[0, :, pl.ds(0, BW)],
                                  wsem.at[j]).wait()

        def compact(t, j):
            iot = _iota16()
            for tb in range(0, BW, 16):
                rowv = tb + iot
                h64 = lax.shift_left(
                    jnp.bitwise_and(idx_v[t, pl.ds(tb, LANES)], 1), 6)

                @plsc.parallel_loop(0, D, step=16)
                def _d(d0, rowv=rowv, h64=h64):
                    for k in range(16):
                        d_vec = d0 + jnp.bitwise_and(iot + k, 15)
                        colv = h64 + d_vec
                        vals = plsc.load_gather(pairs_v.at[j], [rowv, colv])
                        plsc.store_scatter(face_v.at[j], [d_vec, rowv], vals)

        fire_gather(0, 0)

        @pl.loop(0, T)
        def _t(t):
            j = t % 2
            wait_gather(j)

            @pl.when(t + 1 < T)
            def _pf():
                fire_gather(t + 1, 1 - j)

            @pl.when(t >= 2)
            def _ww():
                wait_write(j)

            compact(t, j)
            fire_write(t, j)

        wait_write(0)
        wait_write(1)

    return gather_kernel


def kernel(indices, embed_weight):
    bt, t = indices.shape
    v, d = embed_weight.shape
    tab_t = jnp.transpose(embed_weight)
    idx_t = jnp.transpose(indices)
    wfull = v // 128
    tail = embed_weight[wfull * 128:].reshape(-1, 2 * d)
    tabp = _make_repack(v)(tab_t, tail)
    out_t = _make_gather(t, bt, v // 2)(tabp, idx_t)
    return jnp.transpose(out_t, (2, 0, 1))
